# baseline (speedup 1.0000x reference)
import functools

import jax
import jax.numpy as jnp
from jax import lax
from jax.experimental import pallas as pl
from jax.experimental.pallas import tpu as pltpu
from jax.experimental.pallas import tpu_sc as plsc

D = 64
K = 8192
N = 16384
TN = 1024
TK = 1024

NC = 2
NS = 16
NW = NC * NS
BPW = N // NW
IDX_CH = 128
NCH = BPW // IDX_CH


def _assign_body(z_ref, zz_ref, c_ref, cc_ref, codes_ref):
    zv = z_ref[...]
    zzv = zz_ref[...]
    zh = (2.0 * zv).astype(jnp.bfloat16)

    def step(j, carry):
        ans, g = carry
        cblk = c_ref[pl.ds(j * TK, TK), :]
        ccblk = cc_ref[:, pl.ds(j * TK, TK)]
        chb = cblk.astype(jnp.bfloat16)
        zc2 = lax.dot_general(zh, chb, (((1,), (1,)), ((), ())),
                              preferred_element_type=jnp.float32)
        d2 = (zzv - zc2) + ccblk
        y = d2.astype(jnp.bfloat16).astype(jnp.float32)
        p = y
        s = 1
        while s < TK:
            shifted = jnp.concatenate(
                [jnp.full((TN, s), jnp.inf, jnp.float32), p[:, :TK - s]], axis=1)
            p = jnp.minimum(p, shifted)
            s *= 2
        m_excl = jnp.concatenate(
            [jnp.full((TN, 1), jnp.inf, jnp.float32), p[:, :TK - 1]], axis=1)
        m = jnp.minimum(m_excl, g)
        take = d2 < m
        col = lax.broadcasted_iota(jnp.int32, (TN, TK), 1)
        ti = jnp.max(jnp.where(take, col, -1), axis=1, keepdims=True)
        ans = jnp.where(ti >= 0, ti + j * TK, ans)
        g = jnp.minimum(g, jnp.min(y, axis=1, keepdims=True))
        return ans, g

    ans0 = jnp.zeros((TN, 1), jnp.int32)
    g0 = jnp.full((TN, 1), jnp.inf, jnp.float32)
    ans, _ = lax.fori_loop(0, K // TK, step, (ans0, g0))
    codes_ref[...] = ans


_assign = pl.pallas_call(
    _assign_body,
    grid=(N // TN,),
    in_specs=[
        pl.BlockSpec((TN, D), lambda i: (i, 0)),
        pl.BlockSpec((TN, 1), lambda i: (i, 0)),
        pl.BlockSpec((K, D), lambda i: (0, 0)),
        pl.BlockSpec((1, K), lambda i: (0, 0)),
    ],
    out_specs=pl.BlockSpec((TN, 1), lambda i: (i, 0)),
    out_shape=jax.ShapeDtypeStruct((N, 1), jnp.int32),
)


DP = 128


@functools.lru_cache(maxsize=None)
def _gather_kernel():
    mesh = plsc.VectorSubcoreMesh(core_axis_name="c", subcore_axis_name="s")

    @functools.partial(
        pl.kernel,
        mesh=mesh,
        out_type=jax.ShapeDtypeStruct((N, DP), jnp.float32),
        scratch_types=[
            pltpu.VMEM((BPW,), jnp.int32),
            pltpu.VMEM((BPW, DP), jnp.float32),
            pltpu.SemaphoreType.DMA,
        ],
    )
    def gather(table_hbm, idx_hbm, out_hbm, idx_v, rows_v, sem):
        wid = lax.axis_index("s") * NC + lax.axis_index("c")
        base = wid * BPW
        pltpu.sync_copy(idx_hbm.at[pl.ds(base, BPW)], idx_v)
        copies = []
        for j in range(NCH):
            copies.append(pltpu.async_copy(
                table_hbm.at[idx_v.at[pl.ds(j * IDX_CH, IDX_CH)]],
                rows_v.at[pl.ds(j * IDX_CH, IDX_CH)],
                sem))
        for cp in copies:
            cp.wait()
        pltpu.sync_copy(rows_v, out_hbm.at[pl.ds(base, BPW)])

    return gather


def kernel(z, centers):
    B, T, d = z.shape
    flat = z.reshape(-1, d)
    c = centers / (jnp.linalg.norm(centers, axis=1, keepdims=True) + 1e-8)
    zz = jnp.sum(z * z, axis=2).reshape(-1, 1)
    cc = jnp.sum(c * c, axis=1)[None, :]
    codes = _assign(flat, zz, c, cc)
    table = jnp.pad(c, ((0, 0), (0, DP - D)))
    quant = _gather_kernel()(table, codes.reshape(-1))
    return quant[:, :D].reshape(B, T, d)


# speedup vs baseline: 3.1208x; 3.1208x over previous
import functools

import jax
import jax.numpy as jnp
from jax import lax
from jax.experimental import pallas as pl
from jax.experimental.pallas import tpu as pltpu
from jax.experimental.pallas import tpu_sc as plsc

D = 64
K = 8192
N = 16384
TN = 1024
TK = 1024

NC = 2
NS = 16
NW = NC * NS
BPW = N // NW
IDX_CH = 128
NCH = BPW // IDX_CH


def _assign_body(z_ref, zz_ref, c_ref, cc_ref, codes_ref):
    zv = z_ref[...]
    zzv = zz_ref[...]
    zh = (2.0 * zv).astype(jnp.bfloat16)

    def step(j, carry):
        ans, g = carry
        cblk = c_ref[pl.ds(j * TK, TK), :]
        ccblk = cc_ref[:, pl.ds(j * TK, TK)]
        chb = cblk.astype(jnp.bfloat16)
        zc2 = lax.dot_general(zh, chb, (((1,), (1,)), ((), ())),
                              preferred_element_type=jnp.float32)
        d2 = (zzv - zc2) + ccblk
        lv = jnp.min(d2, axis=1, keepdims=True)
        col = lax.broadcasted_iota(jnp.int32, (TN, TK), 1)
        li = jnp.min(jnp.where(d2 == lv, col, K), axis=1, keepdims=True) + j * TK
        take = lv < g
        ans = jnp.where(take, li, ans)
        g = jnp.where(take, lv.astype(jnp.bfloat16).astype(jnp.float32), g)
        return ans, g

    ans0 = jnp.zeros((TN, 1), jnp.int32)
    g0 = jnp.full((TN, 1), jnp.inf, jnp.float32)
    ans, _ = lax.fori_loop(0, K // TK, step, (ans0, g0))
    codes_ref[...] = ans


_assign = pl.pallas_call(
    _assign_body,
    grid=(N // TN,),
    in_specs=[
        pl.BlockSpec((TN, D), lambda i: (i, 0)),
        pl.BlockSpec((TN, 1), lambda i: (i, 0)),
        pl.BlockSpec((K, D), lambda i: (0, 0)),
        pl.BlockSpec((1, K), lambda i: (0, 0)),
    ],
    out_specs=pl.BlockSpec((TN, 1), lambda i: (i, 0)),
    out_shape=jax.ShapeDtypeStruct((N, 1), jnp.int32),
)


DP = 128


@functools.lru_cache(maxsize=None)
def _gather_kernel():
    mesh = plsc.VectorSubcoreMesh(core_axis_name="c", subcore_axis_name="s")

    @functools.partial(
        pl.kernel,
        mesh=mesh,
        out_type=jax.ShapeDtypeStruct((N, DP), jnp.float32),
        scratch_types=[
            pltpu.VMEM((BPW,), jnp.int32),
            pltpu.VMEM((BPW, DP), jnp.float32),
            pltpu.SemaphoreType.DMA,
        ],
    )
    def gather(table_hbm, idx_hbm, out_hbm, idx_v, rows_v, sem):
        wid = lax.axis_index("s") * NC + lax.axis_index("c")
        base = wid * BPW
        pltpu.sync_copy(idx_hbm.at[pl.ds(base, BPW)], idx_v)
        copies = []
        for j in range(NCH):
            copies.append(pltpu.async_copy(
                table_hbm.at[idx_v.at[pl.ds(j * IDX_CH, IDX_CH)]],
                rows_v.at[pl.ds(j * IDX_CH, IDX_CH)],
                sem))
        for cp in copies:
            cp.wait()
        pltpu.sync_copy(rows_v, out_hbm.at[pl.ds(base, BPW)])

    return gather


def kernel(z, centers):
    B, T, d = z.shape
    flat = z.reshape(-1, d)
    c = centers / (jnp.linalg.norm(centers, axis=1, keepdims=True) + 1e-8)
    zz = jnp.sum(z * z, axis=2).reshape(-1, 1)
    cc = jnp.sum(c * c, axis=1)[None, :]
    codes = _assign(flat, zz, c, cc)
    table = jnp.pad(c, ((0, 0), (0, DP - D)))
    quant = _gather_kernel()(table, codes.reshape(-1))
    return quant[:, :D].reshape(B, T, d)
